# baseline (speedup 1.0000x reference)
import functools

import jax
import jax.numpy as jnp
from jax import lax
from jax.experimental import pallas as pl
from jax.experimental.pallas import tpu as pltpu
from jax.experimental.pallas import tpu_sc as plsc

_NC = 2
_NS = 16
_NW = _NC * _NS
_CSZ = 96


def _pad_n(n):
    return ((n + 127) // 128) * 128


def _edge_tiles(src, dst, n, e):
    np_ = _pad_n(n)
    epw = -(-e // _NW)
    pe = -(-epw // _CSZ) * _CSZ
    if (pe // _CSZ) % 2 == 0:
        pe += _CSZ
    nch = pe // _CSZ
    dump = n + (jnp.arange(_NW, dtype=jnp.int32) % (np_ - n))
    src_p = jnp.pad(src, (0, _NW * epw - e)).reshape(_NW, epw)
    dst_p = jnp.pad(dst, (0, _NW * epw - e),
                    constant_values=np_ - 1).reshape(_NW, epw)
    src_p = jnp.pad(src_p, ((0, 0), (0, pe - epw)))
    dst_p = jnp.concatenate(
        [dst_p, jnp.broadcast_to(dump[:, None], (_NW, pe - epw))], axis=1)
    return src_p.reshape(-1), dst_p.reshape(-1), pe, nch


@functools.partial(jax.jit, static_argnames=("n", "d", "pe", "nch"))
def _sc_segment_sum(h, src_f, dst_f, zeros_tile, *, n, d, pe, nch):
    np_ = _pad_n(n)
    rpt = np_ // _NS
    assert nch % 2 == 1 and nch >= 3

    mesh = plsc.VectorSubcoreMesh(core_axis_name="c", subcore_axis_name="s")

    @functools.partial(
        pl.kernel,
        mesh=mesh,
        out_type=jax.ShapeDtypeStruct((_NC * np_, d), jnp.float32),
        scratch_types=[
            pltpu.VMEM((_CSZ,), jnp.int32),
            pltpu.VMEM((_CSZ,), jnp.int32),
            pltpu.VMEM((_CSZ,), jnp.int32),
            pltpu.VMEM((_CSZ,), jnp.int32),
            pltpu.VMEM((_CSZ, d), jnp.float32),
            pltpu.VMEM((_CSZ, d), jnp.float32),
            pltpu.VMEM_SHARED((np_, d), jnp.float32),
            pltpu.SemaphoreType.DMA,
            pltpu.SemaphoreType.DMA,
            pltpu.SemaphoreType.DMA,
            pltpu.SemaphoreType.DMA,
        ],
    )
    def agg_kernel(h_hbm, src_hbm, dst_hbm, z_hbm, out_hbm,
                   src_a, dst_a, src_b, dst_b, rows_a, rows_b, acc_sh,
                   sem_ia, sem_ib, sem_ga, sem_gb):
        c = lax.axis_index("c")
        s = lax.axis_index("s")
        wid = s * _NC + c
        r0 = s * rpt
        base = wid * pe

        def i_start(j, src_v, dst_v, sem):
            off = base + j * _CSZ
            pltpu.async_copy(src_hbm.at[pl.ds(off, _CSZ)], src_v, sem)
            pltpu.async_copy(dst_hbm.at[pl.ds(off, _CSZ)], dst_v, sem)

        def i_wait(src_v, dst_v, sem):
            pltpu.make_async_copy(src_hbm.at[pl.ds(0, _CSZ)], src_v,
                                  sem).wait()
            pltpu.make_async_copy(dst_hbm.at[pl.ds(0, _CSZ)], dst_v,
                                  sem).wait()

        def g_start(src_v, buf, sem):
            pltpu.async_copy(h_hbm.at[src_v], buf, sem)

        def g_wait(buf, sem):
            pltpu.make_async_copy(h_hbm.at[pl.ds(0, _CSZ)], buf, sem).wait()

        def scat(dst_v, buf):
            pltpu.sync_copy(buf, acc_sh.at[dst_v], add=True)

        i_start(0, src_a, dst_a, sem_ia)
        i_start(1, src_b, dst_b, sem_ib)
        pltpu.sync_copy(z_hbm, acc_sh.at[pl.ds(r0, rpt)])
        i_wait(src_a, dst_a, sem_ia)
        plsc.subcore_barrier()
        g_start(src_a, rows_a, sem_ga)

        def body(i, carry):
            j = 2 * i + 1
            i_wait(src_b, dst_b, sem_ib)
            g_start(src_b, rows_b, sem_gb)
            g_wait(rows_a, sem_ga)
            scat(dst_a, rows_a)
            i_start(j + 1, src_a, dst_a, sem_ia)
            i_wait(src_a, dst_a, sem_ia)
            g_start(src_a, rows_a, sem_ga)
            g_wait(rows_b, sem_gb)
            scat(dst_b, rows_b)

            @pl.when(j + 2 < nch)
            def _():
                i_start(j + 2, src_b, dst_b, sem_ib)

            return carry

        lax.fori_loop(0, (nch - 1) // 2, body, 0)
        g_wait(rows_a, sem_ga)
        scat(dst_a, rows_a)
        plsc.subcore_barrier()
        pltpu.sync_copy(acc_sh.at[pl.ds(r0, rpt)],
                        out_hbm.at[pl.ds(c * np_ + r0, rpt)])

    out = agg_kernel(h, src_f, dst_f, zeros_tile)
    return out.reshape(_NC, np_, d)


@functools.partial(jax.jit, static_argnames=("n", "d"))
def _sc_degree(dst_t, ones_tile, zeros_tile, *, n, d):
    nch = dst_t.shape[1]
    np_ = _pad_n(n)
    rpt = np_ // _NS

    mesh = plsc.VectorSubcoreMesh(core_axis_name="c", subcore_axis_name="s")

    @functools.partial(
        pl.kernel,
        mesh=mesh,
        out_type=jax.ShapeDtypeStruct((_NC * np_, d), jnp.float32),
        scratch_types=[
            pltpu.VMEM((nch, _CSZ), jnp.int32),
            pltpu.VMEM((_CSZ, d), jnp.float32),
            pltpu.VMEM_SHARED((np_, d), jnp.float32),
            pltpu.SemaphoreType.DMA,
        ],
    )
    def deg_kernel(dst_hbm, ones_hbm, z_hbm, out_hbm,
                   dst_v, ones_v, deg_sh, sem):
        c = lax.axis_index("c")
        s = lax.axis_index("s")
        wid = s * _NC + c
        r0 = s * rpt
        cp = pltpu.async_copy(dst_hbm.at[wid], dst_v, sem)
        pltpu.sync_copy(z_hbm, deg_sh.at[pl.ds(r0, rpt)])
        pltpu.sync_copy(ones_hbm, ones_v)
        cp.wait()
        plsc.subcore_barrier()

        def body(i, carry):
            pltpu.sync_copy(ones_v, deg_sh.at[dst_v.at[i]], add=True)
            return carry

        lax.fori_loop(0, nch, body, 0)
        plsc.subcore_barrier()
        pltpu.sync_copy(deg_sh.at[pl.ds(r0, rpt)],
                        out_hbm.at[pl.ds(c * np_ + r0, rpt)])

    out = deg_kernel(dst_t, ones_tile, zeros_tile)
    return out.reshape(_NC, np_, d)[:, :n, 0:1]


def _tc_layer_bn_relu(P, degp, h, Wl, bl, Wr, g, beta):
    n, d = h.shape

    def body(p_ref, degp_ref, h_ref, wl_ref, bl_ref, wr_ref, g_ref, b_ref,
             o_ref):
        deg = degp_ref[0] + degp_ref[1]
        inv = 1.0 / jnp.maximum(deg, 1.0)
        mean = (p_ref[0, :n, :] + p_ref[1, :n, :]) * inv
        z = (jnp.dot(mean, wl_ref[...], preferred_element_type=jnp.float32)
             + bl_ref[...]
             + jnp.dot(h_ref[...], wr_ref[...],
                       preferred_element_type=jnp.float32))
        mu = jnp.mean(z, axis=0, keepdims=True)
        var = jnp.mean((z - mu) * (z - mu), axis=0, keepdims=True)
        zn = g_ref[...] * (z - mu) * lax.rsqrt(var + 1e-5) + b_ref[...]
        o_ref[...] = jnp.maximum(zn, 0.0)

    return pl.pallas_call(
        body,
        out_shape=jax.ShapeDtypeStruct((n, d), jnp.float32),
    )(P, degp, h, Wl, bl.reshape(1, -1), Wr, g.reshape(1, -1),
      beta.reshape(1, -1))


def _tc_layer_final(P, degp, h, Wl, bl, Wr, Wp, bp):
    n, d = h.shape
    o = Wp.shape[1]

    def body(p_ref, degp_ref, h_ref, wl_ref, bl_ref, wr_ref, wp_ref, bp_ref,
             out_ref):
        deg = degp_ref[0] + degp_ref[1]
        inv = 1.0 / jnp.maximum(deg, 1.0)
        mean = (p_ref[0, :n, :] + p_ref[1, :n, :]) * inv
        z = (jnp.dot(mean, wl_ref[...], preferred_element_type=jnp.float32)
             + bl_ref[...]
             + jnp.dot(h_ref[...], wr_ref[...],
                       preferred_element_type=jnp.float32))
        logits = jnp.dot(z, wp_ref[...],
                         preferred_element_type=jnp.float32) + bp_ref[...]
        m = jnp.max(logits, axis=1, keepdims=True)
        shifted = logits - m
        lse = jnp.log(jnp.sum(jnp.exp(shifted), axis=1, keepdims=True))
        out_ref[...] = shifted - lse

    return pl.pallas_call(
        body,
        out_shape=jax.ShapeDtypeStruct((n, o), jnp.float32),
    )(P, degp, h, Wl, bl.reshape(1, -1), Wr, Wp, bp.reshape(1, -1))


def kernel(x, edge_index, Wl0, bl0, Wr0, Wl1, bl1, Wr1, Wl2, bl2, Wr2,
           g0, beta0, g1, beta1, Wp, bp):
    n, d = x.shape
    e = edge_index.shape[1]
    src = edge_index[0]
    dst = edge_index[1]

    src_f, dst_f, pe, nch = _edge_tiles(src, dst, n, e)

    rpt = _pad_n(n) // _NS
    zeros_tile = jnp.zeros((rpt, d), jnp.float32)
    ones_tile = jnp.ones((_CSZ, d), jnp.float32)

    degp = _sc_degree(dst_f.reshape(_NW, nch, _CSZ), ones_tile, zeros_tile,
                      n=n, d=d)

    P0 = _sc_segment_sum(x, src_f, dst_f, zeros_tile, n=n, d=d, pe=pe,
                         nch=nch)
    h1 = _tc_layer_bn_relu(P0, degp, x, Wl0, bl0, Wr0, g0, beta0)

    P1 = _sc_segment_sum(h1, src_f, dst_f, zeros_tile, n=n, d=d, pe=pe,
                         nch=nch)
    h2 = _tc_layer_bn_relu(P1, degp, h1, Wl1, bl1, Wr1, g1, beta1)

    P2 = _sc_segment_sum(h2, src_f, dst_f, zeros_tile, n=n, d=d, pe=pe,
                         nch=nch)
    return _tc_layer_final(P2, degp, h2, Wl2, bl2, Wr2, Wp, bp)


# speedup vs baseline: 1.4077x; 1.4077x over previous
import functools

import jax
import jax.numpy as jnp
from jax import lax
from jax.experimental import pallas as pl
from jax.experimental.pallas import tpu as pltpu
from jax.experimental.pallas import tpu_sc as plsc

_NC = 2
_NS = 16
_NW = _NC * _NS
_CSZ = 80


def _pad_n(n):
    return ((n + 127) // 128) * 128


def _edge_tiles(src, dst, n, e):
    np_ = _pad_n(n)
    epw = -(-e // _NW)
    pe = -(-epw // _CSZ) * _CSZ
    if (pe // _CSZ) % 2 == 0:
        pe += _CSZ
    nch = pe // _CSZ
    dump = n + (jnp.arange(_NW, dtype=jnp.int32) % (np_ - n))
    src_p = jnp.pad(src, (0, _NW * epw - e)).reshape(_NW, epw)
    dst_p = jnp.pad(dst, (0, _NW * epw - e),
                    constant_values=np_ - 1).reshape(_NW, epw)
    src_p = jnp.pad(src_p, ((0, 0), (0, pe - epw)))
    dst_p = jnp.concatenate(
        [dst_p, jnp.broadcast_to(dump[:, None], (_NW, pe - epw))], axis=1)
    return src_p.reshape(-1), dst_p.reshape(-1), pe, nch


@functools.partial(jax.jit, static_argnames=("n", "d", "pe", "nch"))
def _sc_segment_sum(h, src_f, dst_f, zeros_tile, *, n, d, pe, nch):
    np_ = _pad_n(n)
    rpt = np_ // _NS
    assert nch % 2 == 1 and nch >= 3

    mesh = plsc.VectorSubcoreMesh(core_axis_name="c", subcore_axis_name="s")

    @functools.partial(
        pl.kernel,
        mesh=mesh,
        out_type=jax.ShapeDtypeStruct((_NC * np_, d), jnp.float32),
        scratch_types=[
            pltpu.VMEM((_CSZ,), jnp.int32),
            pltpu.VMEM((_CSZ,), jnp.int32),
            pltpu.VMEM((_CSZ,), jnp.int32),
            pltpu.VMEM((_CSZ,), jnp.int32),
            pltpu.VMEM((_CSZ, d), jnp.float32),
            pltpu.VMEM((_CSZ, d), jnp.float32),
            pltpu.VMEM_SHARED((np_, d), jnp.float32),
            pltpu.SemaphoreType.DMA,
            pltpu.SemaphoreType.DMA,
            pltpu.SemaphoreType.DMA,
            pltpu.SemaphoreType.DMA,
        ],
    )
    def agg_kernel(h_hbm, src_hbm, dst_hbm, z_hbm, out_hbm,
                   src_a, dst_a, src_b, dst_b, rows_a, rows_b, acc_sh,
                   sem_ia, sem_ib, sem_ga, sem_gb):
        c = lax.axis_index("c")
        s = lax.axis_index("s")
        wid = s * _NC + c
        r0 = s * rpt
        base = wid * pe

        def i_start(j, src_v, dst_v, sem):
            off = base + j * _CSZ
            pltpu.async_copy(src_hbm.at[pl.ds(off, _CSZ)], src_v, sem)
            pltpu.async_copy(dst_hbm.at[pl.ds(off, _CSZ)], dst_v, sem)

        def i_wait(src_v, dst_v, sem):
            pltpu.make_async_copy(src_hbm.at[pl.ds(0, _CSZ)], src_v,
                                  sem).wait()
            pltpu.make_async_copy(dst_hbm.at[pl.ds(0, _CSZ)], dst_v,
                                  sem).wait()

        def g_start(src_v, buf, sem):
            pltpu.async_copy(h_hbm.at[src_v], buf, sem)

        def g_wait(buf, sem):
            pltpu.make_async_copy(h_hbm.at[pl.ds(0, _CSZ)], buf, sem).wait()

        def scat(dst_v, buf):
            pltpu.sync_copy(buf, acc_sh.at[dst_v], add=True)

        i_start(0, src_a, dst_a, sem_ia)
        i_start(1, src_b, dst_b, sem_ib)
        pltpu.sync_copy(z_hbm, acc_sh.at[pl.ds(r0, rpt)])
        i_wait(src_a, dst_a, sem_ia)
        plsc.subcore_barrier()
        g_start(src_a, rows_a, sem_ga)

        def body(i, carry):
            j = 2 * i + 1
            i_wait(src_b, dst_b, sem_ib)
            g_start(src_b, rows_b, sem_gb)
            g_wait(rows_a, sem_ga)
            scat(dst_a, rows_a)
            i_start(j + 1, src_a, dst_a, sem_ia)
            i_wait(src_a, dst_a, sem_ia)
            g_start(src_a, rows_a, sem_ga)
            g_wait(rows_b, sem_gb)
            scat(dst_b, rows_b)

            @pl.when(j + 2 < nch)
            def _():
                i_start(j + 2, src_b, dst_b, sem_ib)

            return carry

        lax.fori_loop(0, (nch - 1) // 2, body, 0)
        g_wait(rows_a, sem_ga)
        scat(dst_a, rows_a)
        plsc.subcore_barrier()
        pltpu.sync_copy(acc_sh.at[pl.ds(r0, rpt)],
                        out_hbm.at[pl.ds(c * np_ + r0, rpt)])

    out = agg_kernel(h, src_f, dst_f, zeros_tile)
    return out.reshape(_NC, np_, d)


@functools.partial(jax.jit, static_argnames=("n", "d"))
def _sc_degree(dst_t, ones_tile, zeros_tile, *, n, d):
    nch = dst_t.shape[1]
    np_ = _pad_n(n)
    rpt = np_ // _NS

    mesh = plsc.VectorSubcoreMesh(core_axis_name="c", subcore_axis_name="s")

    @functools.partial(
        pl.kernel,
        mesh=mesh,
        out_type=jax.ShapeDtypeStruct((_NC * np_, d), jnp.float32),
        scratch_types=[
            pltpu.VMEM((nch, _CSZ), jnp.int32),
            pltpu.VMEM((_CSZ, d), jnp.float32),
            pltpu.VMEM_SHARED((np_, d), jnp.float32),
            pltpu.SemaphoreType.DMA,
        ],
    )
    def deg_kernel(dst_hbm, ones_hbm, z_hbm, out_hbm,
                   dst_v, ones_v, deg_sh, sem):
        c = lax.axis_index("c")
        s = lax.axis_index("s")
        wid = s * _NC + c
        r0 = s * rpt
        cp = pltpu.async_copy(dst_hbm.at[wid], dst_v, sem)
        pltpu.sync_copy(z_hbm, deg_sh.at[pl.ds(r0, rpt)])
        pltpu.sync_copy(ones_hbm, ones_v)
        cp.wait()
        plsc.subcore_barrier()

        def body(i, carry):
            pltpu.sync_copy(ones_v, deg_sh.at[dst_v.at[i]], add=True)
            return carry

        lax.fori_loop(0, nch, body, 0)
        plsc.subcore_barrier()
        pltpu.sync_copy(deg_sh.at[pl.ds(r0, rpt)],
                        out_hbm.at[pl.ds(c * np_ + r0, rpt)])

    out = deg_kernel(dst_t, ones_tile, zeros_tile)
    return out.reshape(_NC, np_, d)[:, :n, 0:1]


def _tc_layer_bn_relu(P, degp, h, Wl, bl, Wr, g, beta):
    n, d = h.shape

    def body(p_ref, degp_ref, h_ref, wl_ref, bl_ref, wr_ref, g_ref, b_ref,
             o_ref):
        deg = degp_ref[0] + degp_ref[1]
        inv = 1.0 / jnp.maximum(deg, 1.0)
        mean = (p_ref[0, :n, :] + p_ref[1, :n, :]) * inv
        z = (jnp.dot(mean, wl_ref[...], preferred_element_type=jnp.float32)
             + bl_ref[...]
             + jnp.dot(h_ref[...], wr_ref[...],
                       preferred_element_type=jnp.float32))
        mu = jnp.mean(z, axis=0, keepdims=True)
        var = jnp.mean((z - mu) * (z - mu), axis=0, keepdims=True)
        zn = g_ref[...] * (z - mu) * lax.rsqrt(var + 1e-5) + b_ref[...]
        o_ref[...] = jnp.maximum(zn, 0.0)

    return pl.pallas_call(
        body,
        out_shape=jax.ShapeDtypeStruct((n, d), jnp.float32),
    )(P, degp, h, Wl, bl.reshape(1, -1), Wr, g.reshape(1, -1),
      beta.reshape(1, -1))


def _tc_layer_final(P, degp, h, Wl, bl, Wr, Wp, bp):
    n, d = h.shape
    o = Wp.shape[1]

    def body(p_ref, degp_ref, h_ref, wl_ref, bl_ref, wr_ref, wp_ref, bp_ref,
             out_ref):
        deg = degp_ref[0] + degp_ref[1]
        inv = 1.0 / jnp.maximum(deg, 1.0)
        mean = (p_ref[0, :n, :] + p_ref[1, :n, :]) * inv
        z = (jnp.dot(mean, wl_ref[...], preferred_element_type=jnp.float32)
             + bl_ref[...]
             + jnp.dot(h_ref[...], wr_ref[...],
                       preferred_element_type=jnp.float32))
        logits = jnp.dot(z, wp_ref[...],
                         preferred_element_type=jnp.float32) + bp_ref[...]
        m = jnp.max(logits, axis=1, keepdims=True)
        shifted = logits - m
        lse = jnp.log(jnp.sum(jnp.exp(shifted), axis=1, keepdims=True))
        out_ref[...] = shifted - lse

    return pl.pallas_call(
        body,
        out_shape=jax.ShapeDtypeStruct((n, o), jnp.float32),
    )(P, degp, h, Wl, bl.reshape(1, -1), Wr, Wp, bp.reshape(1, -1))


def kernel(x, edge_index, Wl0, bl0, Wr0, Wl1, bl1, Wr1, Wl2, bl2, Wr2,
           g0, beta0, g1, beta1, Wp, bp):
    n, d = x.shape
    e = edge_index.shape[1]
    src = edge_index[0]
    dst = edge_index[1]

    src_f, dst_f, pe, nch = _edge_tiles(src, dst, n, e)

    rpt = _pad_n(n) // _NS
    zeros_tile = jnp.zeros((rpt, d), jnp.float32)
    ones_tile = jnp.ones((_CSZ, d), jnp.float32)

    degp = _sc_degree(dst_f.reshape(_NW, nch, _CSZ), ones_tile, zeros_tile,
                      n=n, d=d)

    P0 = _sc_segment_sum(x, src_f, dst_f, zeros_tile, n=n, d=d, pe=pe,
                         nch=nch)
    h1 = _tc_layer_bn_relu(P0, degp, x, Wl0, bl0, Wr0, g0, beta0)

    P1 = _sc_segment_sum(h1, src_f, dst_f, zeros_tile, n=n, d=d, pe=pe,
                         nch=nch)
    h2 = _tc_layer_bn_relu(P1, degp, h1, Wl1, bl1, Wr1, g1, beta1)

    P2 = _sc_segment_sum(h2, src_f, dst_f, zeros_tile, n=n, d=d, pe=pe,
                         nch=nch)
    return _tc_layer_final(P2, degp, h2, Wl2, bl2, Wr2, Wp, bp)


# speedup vs baseline: 1.4095x; 1.0013x over previous
import functools

import jax
import jax.numpy as jnp
from jax import lax
from jax.experimental import pallas as pl
from jax.experimental.pallas import tpu as pltpu
from jax.experimental.pallas import tpu_sc as plsc

_NC = 2
_NS = 16
_NW = _NC * _NS
_CSZ = 80


def _pad_n(n):
    return ((n + 127) // 128) * 128


def _edge_tiles(src, dst, n, e):
    np_ = _pad_n(n)
    epw = -(-e // _NW)
    pe = -(-epw // _CSZ) * _CSZ
    if (pe // _CSZ) % 2 == 0:
        pe += _CSZ
    nch = pe // _CSZ
    dump = n + (jnp.arange(_NW, dtype=jnp.int32) % (np_ - n))
    src_p = jnp.pad(src, (0, _NW * epw - e)).reshape(_NW, epw)
    dst_p = jnp.pad(dst, (0, _NW * epw - e),
                    constant_values=np_ - 1).reshape(_NW, epw)
    src_p = jnp.pad(src_p, ((0, 0), (0, pe - epw)))
    dst_p = jnp.concatenate(
        [dst_p, jnp.broadcast_to(dump[:, None], (_NW, pe - epw))], axis=1)
    return src_p.reshape(-1), dst_p.reshape(-1), pe, nch


@functools.partial(jax.jit, static_argnames=("n", "d", "pe", "nch"))
def _sc_segment_sum(h, src_f, dst_f, zeros_tile, *, n, d, pe, nch):
    np_ = _pad_n(n)
    rpt = np_ // _NS
    assert nch % 2 == 1 and nch >= 3

    mesh = plsc.VectorSubcoreMesh(core_axis_name="c", subcore_axis_name="s")

    @functools.partial(
        pl.kernel,
        mesh=mesh,
        out_type=jax.ShapeDtypeStruct((_NC * np_, d), jnp.float32),
        scratch_types=[
            pltpu.VMEM((_CSZ,), jnp.int32),
            pltpu.VMEM((_CSZ,), jnp.int32),
            pltpu.VMEM((_CSZ,), jnp.int32),
            pltpu.VMEM((_CSZ,), jnp.int32),
            pltpu.VMEM((_CSZ, d), jnp.float32),
            pltpu.VMEM((_CSZ, d), jnp.float32),
            pltpu.VMEM_SHARED((np_, d), jnp.float32),
            pltpu.SemaphoreType.DMA,
            pltpu.SemaphoreType.DMA,
            pltpu.SemaphoreType.DMA,
            pltpu.SemaphoreType.DMA,
        ],
    )
    def agg_kernel(h_hbm, src_hbm, dst_hbm, z_hbm, out_hbm,
                   src_a, dst_a, src_b, dst_b, rows_a, rows_b, acc_sh,
                   sem_ia, sem_ib, sem_ga, sem_gb):
        c = lax.axis_index("c")
        s = lax.axis_index("s")
        wid = s * _NC + c
        r0 = s * rpt
        base = wid * pe

        def i_start(j, src_v, dst_v, sem):
            off = base + j * _CSZ
            pltpu.async_copy(src_hbm.at[pl.ds(off, _CSZ)], src_v, sem)
            pltpu.async_copy(dst_hbm.at[pl.ds(off, _CSZ)], dst_v, sem)

        def i_wait(src_v, dst_v, sem):
            pltpu.make_async_copy(src_hbm.at[pl.ds(0, _CSZ)], src_v,
                                  sem).wait()
            pltpu.make_async_copy(dst_hbm.at[pl.ds(0, _CSZ)], dst_v,
                                  sem).wait()

        def g_start(src_v, buf, sem):
            pltpu.async_copy(h_hbm.at[src_v], buf, sem)

        def g_wait(buf, sem):
            pltpu.make_async_copy(h_hbm.at[pl.ds(0, _CSZ)], buf, sem).wait()

        def scat(dst_v, buf):
            pltpu.sync_copy(buf, acc_sh.at[dst_v], add=True)

        i_start(0, src_a, dst_a, sem_ia)
        i_start(1, src_b, dst_b, sem_ib)
        pltpu.sync_copy(z_hbm, acc_sh.at[pl.ds(r0, rpt)])
        i_wait(src_a, dst_a, sem_ia)
        plsc.subcore_barrier()
        g_start(src_a, rows_a, sem_ga)

        def body(i, carry):
            j = 2 * i + 1
            i_wait(src_b, dst_b, sem_ib)
            g_start(src_b, rows_b, sem_gb)
            g_wait(rows_a, sem_ga)
            scat(dst_a, rows_a)
            i_start(j + 1, src_a, dst_a, sem_ia)
            i_wait(src_a, dst_a, sem_ia)
            g_start(src_a, rows_a, sem_ga)
            g_wait(rows_b, sem_gb)
            scat(dst_b, rows_b)

            @pl.when(j + 2 < nch)
            def _():
                i_start(j + 2, src_b, dst_b, sem_ib)

            return carry

        lax.fori_loop(0, (nch - 1) // 2, body, 0)
        g_wait(rows_a, sem_ga)
        scat(dst_a, rows_a)
        plsc.subcore_barrier()
        pltpu.sync_copy(acc_sh.at[pl.ds(r0, rpt)],
                        out_hbm.at[pl.ds(c * np_ + r0, rpt)])

    out = agg_kernel(h, src_f, dst_f, zeros_tile)
    return out.reshape(_NC, np_, d)


@functools.partial(jax.jit, static_argnames=("n", "d"))
def _sc_degree(dst_t, ones_tile, zeros_tile, *, n, d):
    nch = dst_t.shape[1]
    np_ = _pad_n(n)
    rpt = np_ // _NS

    mesh = plsc.VectorSubcoreMesh(core_axis_name="c", subcore_axis_name="s")

    @functools.partial(
        pl.kernel,
        mesh=mesh,
        out_type=jax.ShapeDtypeStruct((_NC * np_, d), jnp.float32),
        scratch_types=[
            pltpu.VMEM((nch, _CSZ), jnp.int32),
            pltpu.VMEM((_CSZ, d), jnp.float32),
            pltpu.VMEM_SHARED((np_, d), jnp.float32),
            pltpu.SemaphoreType.DMA,
            pltpu.SemaphoreType.DMA,
        ],
    )
    def deg_kernel(dst_hbm, ones_hbm, z_hbm, out_hbm,
                   dst_v, ones_v, deg_sh, sem_a, sem_b):
        c = lax.axis_index("c")
        s = lax.axis_index("s")
        wid = s * _NC + c
        r0 = s * rpt
        cp = pltpu.async_copy(dst_hbm.at[wid], dst_v, sem_a)
        pltpu.sync_copy(z_hbm, deg_sh.at[pl.ds(r0, rpt)])
        pltpu.sync_copy(ones_hbm, ones_v)
        cp.wait()
        plsc.subcore_barrier()

        def scat_start(j, sem):
            pltpu.async_copy(ones_v, deg_sh.at[dst_v.at[j]], sem, add=True)

        def scat_wait(sem):
            pltpu.make_async_copy(ones_hbm, ones_v, sem).wait()

        scat_start(0, sem_a)

        def body(i, carry):
            scat_start(2 * i + 1, sem_b)
            scat_wait(sem_a)
            scat_start(2 * i + 2, sem_a)
            scat_wait(sem_b)
            return carry

        lax.fori_loop(0, (nch - 1) // 2, body, 0)
        scat_wait(sem_a)
        plsc.subcore_barrier()
        pltpu.sync_copy(deg_sh.at[pl.ds(r0, rpt)],
                        out_hbm.at[pl.ds(c * np_ + r0, rpt)])

    out = deg_kernel(dst_t, ones_tile, zeros_tile)
    return out.reshape(_NC, np_, d)[:, :n, 0:1]


def _tc_layer_bn_relu(P, degp, h, Wl, bl, Wr, g, beta):
    n, d = h.shape

    def body(p_ref, degp_ref, h_ref, wl_ref, bl_ref, wr_ref, g_ref, b_ref,
             o_ref):
        deg = degp_ref[0] + degp_ref[1]
        inv = 1.0 / jnp.maximum(deg, 1.0)
        mean = (p_ref[0, :n, :] + p_ref[1, :n, :]) * inv
        z = (jnp.dot(mean, wl_ref[...], preferred_element_type=jnp.float32)
             + bl_ref[...]
             + jnp.dot(h_ref[...], wr_ref[...],
                       preferred_element_type=jnp.float32))
        mu = jnp.mean(z, axis=0, keepdims=True)
        var = jnp.mean((z - mu) * (z - mu), axis=0, keepdims=True)
        zn = g_ref[...] * (z - mu) * lax.rsqrt(var + 1e-5) + b_ref[...]
        o_ref[...] = jnp.maximum(zn, 0.0)

    return pl.pallas_call(
        body,
        out_shape=jax.ShapeDtypeStruct((n, d), jnp.float32),
    )(P, degp, h, Wl, bl.reshape(1, -1), Wr, g.reshape(1, -1),
      beta.reshape(1, -1))


def _tc_layer_final(P, degp, h, Wl, bl, Wr, Wp, bp):
    n, d = h.shape
    o = Wp.shape[1]

    def body(p_ref, degp_ref, h_ref, wl_ref, bl_ref, wr_ref, wp_ref, bp_ref,
             out_ref):
        deg = degp_ref[0] + degp_ref[1]
        inv = 1.0 / jnp.maximum(deg, 1.0)
        mean = (p_ref[0, :n, :] + p_ref[1, :n, :]) * inv
        z = (jnp.dot(mean, wl_ref[...], preferred_element_type=jnp.float32)
             + bl_ref[...]
             + jnp.dot(h_ref[...], wr_ref[...],
                       preferred_element_type=jnp.float32))
        logits = jnp.dot(z, wp_ref[...],
                         preferred_element_type=jnp.float32) + bp_ref[...]
        m = jnp.max(logits, axis=1, keepdims=True)
        shifted = logits - m
        lse = jnp.log(jnp.sum(jnp.exp(shifted), axis=1, keepdims=True))
        out_ref[...] = shifted - lse

    return pl.pallas_call(
        body,
        out_shape=jax.ShapeDtypeStruct((n, o), jnp.float32),
    )(P, degp, h, Wl, bl.reshape(1, -1), Wr, Wp, bp.reshape(1, -1))


def kernel(x, edge_index, Wl0, bl0, Wr0, Wl1, bl1, Wr1, Wl2, bl2, Wr2,
           g0, beta0, g1, beta1, Wp, bp):
    n, d = x.shape
    e = edge_index.shape[1]
    src = edge_index[0]
    dst = edge_index[1]

    src_f, dst_f, pe, nch = _edge_tiles(src, dst, n, e)

    rpt = _pad_n(n) // _NS
    zeros_tile = jnp.zeros((rpt, d), jnp.float32)
    ones_tile = jnp.ones((_CSZ, d), jnp.float32)

    degp = _sc_degree(dst_f.reshape(_NW, nch, _CSZ), ones_tile, zeros_tile,
                      n=n, d=d)

    P0 = _sc_segment_sum(x, src_f, dst_f, zeros_tile, n=n, d=d, pe=pe,
                         nch=nch)
    h1 = _tc_layer_bn_relu(P0, degp, x, Wl0, bl0, Wr0, g0, beta0)

    P1 = _sc_segment_sum(h1, src_f, dst_f, zeros_tile, n=n, d=d, pe=pe,
                         nch=nch)
    h2 = _tc_layer_bn_relu(P1, degp, h1, Wl1, bl1, Wr1, g1, beta1)

    P2 = _sc_segment_sum(h2, src_f, dst_f, zeros_tile, n=n, d=d, pe=pe,
                         nch=nch)
    return _tc_layer_final(P2, degp, h2, Wl2, bl2, Wr2, Wp, bp)
